# baseline (speedup 1.0000x reference)
import functools

import jax
import jax.numpy as jnp
from jax import lax
from jax.experimental import pallas as pl
from jax.experimental.pallas import tpu as pltpu
from jax.experimental.pallas import tpu_sc as plsc

_NC = 2
_NS = 16
_NW = _NC * _NS
_LANES = 16
_CH = 8
_HALF = 32
_NBUF = 3


def kernel(x, pos_embedding):
    b, s, d = x.shape
    rows_per_w = s // _NW
    n_chunks = rows_per_w // _CH
    n_loop = n_chunks - 2
    n_iters = n_loop // _NBUF
    groups = d // _LANES
    mesh = plsc.VectorSubcoreMesh(core_axis_name="c", subcore_axis_name="s")

    @functools.partial(
        pl.kernel,
        mesh=mesh,
        out_type=jax.ShapeDtypeStruct((b, s, d), jnp.float32),
        scratch_types=[
            pltpu.VMEM((_NBUF, _CH, d), jnp.float32),
            pltpu.VMEM((_NBUF, b, _CH, d), jnp.float32),
            pltpu.SemaphoreType.DMA((_NBUF,)),
            pltpu.SemaphoreType.DMA((_NBUF,)),
            pltpu.SemaphoreType.DMA((_NBUF,)),
        ],
    )
    def k(x_hbm, pos_hbm, out_hbm, pos_v, x_v, sem_in, sem_out, sem_pos):
        wid = lax.axis_index("s") * _NC + lax.axis_index("c")
        row_base = wid * rows_per_w

        def add_and_store(bf, rows):
            def row_body(r, _):
                for h in range(groups // _HALF):
                    base = h * _HALF * _LANES
                    pos_regs = [
                        pos_v[bf, r, pl.ds(base + g * _LANES, _LANES)]
                        for g in range(_HALF)
                    ]
                    for bi in range(b):
                        for g in range(_HALF):
                            o = base + g * _LANES
                            x_v[bf, bi, r, pl.ds(o, _LANES)] = (
                                x_v[bf, bi, r, pl.ds(o, _LANES)]
                                + pos_regs[g])
                return 0

            hc = _CH // 2
            for hh in range(2):
                lax.fori_loop(hh * hc, (hh + 1) * hc, row_body, 0)
                pltpu.async_copy(
                    x_v.at[bf, :, pl.ds(hh * hc, hc)],
                    out_hbm.at[:, pl.ds(rows + hh * hc, hc)],
                    sem_out.at[bf])

        def drain_out(bf, rows):
            pltpu.make_async_copy(
                x_v.at[bf], out_hbm.at[:, pl.ds(rows, _CH)],
                sem_out.at[bf]).wait()

        def issue_in(bf, rows):
            pltpu.async_copy(x_hbm.at[:, pl.ds(rows, _CH)], x_v.at[bf],
                             sem_in.at[bf])

        def issue_pos(bf, rows):
            pltpu.async_copy(pos_hbm.at[pl.ds(rows, _CH)], pos_v.at[bf],
                             sem_pos.at[bf])

        def wait_in(bf, rows):
            pltpu.make_async_copy(
                x_hbm.at[:, pl.ds(rows, _CH)], x_v.at[bf],
                sem_in.at[bf]).wait()
            pltpu.make_async_copy(
                pos_hbm.at[pl.ds(rows, _CH)], pos_v.at[bf],
                sem_pos.at[bf]).wait()

        issue_pos(0, row_base)
        issue_in(0, row_base)

        def iter_body(cp, _):
            for k_ in range(_NBUF):
                rows = row_base + (cp * _NBUF + k_) * _CH
                nb = (k_ + 1) % _NBUF
                issue_pos(nb, rows + _CH)
                if k_ < 2:
                    @pl.when(cp > 0)
                    def _():
                        drain_out(nb, rows - 2 * _CH)
                else:
                    drain_out(nb, rows - 2 * _CH)
                issue_in(nb, rows + _CH)
                wait_in(k_, rows)
                add_and_store(k_, rows)
            return 0

        lax.fori_loop(0, n_iters, iter_body, 0)

        r30 = row_base + n_loop * _CH
        r31 = r30 + _CH
        issue_pos(1, r31)
        drain_out(1, r30 - 2 * _CH)
        issue_in(1, r31)
        wait_in(0, r30)
        add_and_store(0, r30)

        drain_out(2, r31 - 2 * _CH)
        wait_in(1, r31)
        add_and_store(1, r31)

        drain_out(0, r30)
        drain_out(1, r31)

    return k(x, pos_embedding)


# speedup vs baseline: 1.0345x; 1.0345x over previous
import functools

import jax
import jax.numpy as jnp
from jax import lax
from jax.experimental import pallas as pl
from jax.experimental.pallas import tpu as pltpu
from jax.experimental.pallas import tpu_sc as plsc

_NC = 2
_NS = 16
_NW = _NC * _NS
_LANES = 16
_CH = 8
_HALF = 32


def kernel(x, pos_embedding):
    b, s, d = x.shape
    rows_per_w = s // _NW
    n_chunks = rows_per_w // _CH
    n_iters = n_chunks // 2
    groups = d // _LANES
    mesh = plsc.VectorSubcoreMesh(core_axis_name="c", subcore_axis_name="s")

    @functools.partial(
        pl.kernel,
        mesh=mesh,
        out_type=jax.ShapeDtypeStruct((b, s, d), jnp.float32),
        scratch_types=[
            pltpu.VMEM((2, _CH, d), jnp.float32),
            pltpu.VMEM((2, b, _CH, d), jnp.float32),
            pltpu.SemaphoreType.DMA((2, 2)),
            pltpu.SemaphoreType.DMA((2, 2)),
            pltpu.SemaphoreType.DMA((2,)),
        ],
    )
    def k(x_hbm, pos_hbm, out_hbm, pos_v, x_v, sem_in, sem_out, sem_pos):
        wid = lax.axis_index("s") * _NC + lax.axis_index("c")
        row_base = wid * rows_per_w
        hc = _CH // 2

        def in_half(bf, rows, hh):
            return pltpu.make_async_copy(
                x_hbm.at[:, pl.ds(rows + hh * hc, hc)],
                x_v.at[bf, :, pl.ds(hh * hc, hc)], sem_in.at[bf, hh])

        def out_half(bf, rows, hh):
            return pltpu.make_async_copy(
                x_v.at[bf, :, pl.ds(hh * hc, hc)],
                out_hbm.at[:, pl.ds(rows + hh * hc, hc)], sem_out.at[bf, hh])

        pltpu.async_copy(pos_hbm.at[pl.ds(row_base, _CH)], pos_v.at[0],
                         sem_pos.at[0])
        in_half(0, row_base, 0).start()
        in_half(0, row_base, 1).start()

        def iter_body(cp, _):
            for k_ in range(2):
                xb = k_
                ob = (k_ + 1) % 2
                rows = row_base + (cp * 2 + k_) * _CH
                nxt = rows + _CH

                pltpu.async_copy(pos_hbm.at[pl.ds(nxt, _CH)],
                                 pos_v.at[ob], sem_pos.at[ob])

                if k_ == 0:
                    @pl.when(cp > 0)
                    def _():
                        out_half(ob, rows - _CH, 0).wait()
                    in_half(ob, nxt, 0).start()

                    @pl.when(cp > 0)
                    def _():
                        out_half(ob, rows - _CH, 1).wait()
                    in_half(ob, nxt, 1).start()
                else:
                    out_half(ob, rows - _CH, 0).wait()

                    @pl.when(cp + 1 < n_iters)
                    def _():
                        in_half(ob, nxt, 0).start()
                    out_half(ob, rows - _CH, 1).wait()

                    @pl.when(cp + 1 < n_iters)
                    def _():
                        in_half(ob, nxt, 1).start()

                in_half(xb, rows, 0).wait()
                in_half(xb, rows, 1).wait()
                pltpu.make_async_copy(
                    pos_hbm.at[pl.ds(rows, _CH)], pos_v.at[xb],
                    sem_pos.at[xb]).wait()

                def row_body(r, _):
                    for h in range(groups // _HALF):
                        base = h * _HALF * _LANES
                        pos_regs = [
                            pos_v[xb, r, pl.ds(base + g * _LANES, _LANES)]
                            for g in range(_HALF)
                        ]
                        for bi in range(b):
                            for g in range(_HALF):
                                o = base + g * _LANES
                                x_v[xb, bi, r, pl.ds(o, _LANES)] = (
                                    x_v[xb, bi, r, pl.ds(o, _LANES)]
                                    + pos_regs[g])
                    return 0

                for hh in range(2):
                    lax.fori_loop(hh * hc, (hh + 1) * hc, row_body, 0)
                    out_half(xb, rows, hh).start()
            return 0

        lax.fori_loop(0, n_iters, iter_body, 0)

        last = row_base + (n_chunks - 1) * _CH
        out_half(1, last, 0).wait()
        out_half(1, last, 1).wait()
        pltpu.make_async_copy(
            pos_hbm.at[pl.ds(row_base, _CH)], pos_v.at[0],
            sem_pos.at[0]).wait()

    return k(x, pos_embedding)


# speedup vs baseline: 1.0421x; 1.0073x over previous
import functools

import jax
import jax.numpy as jnp
from jax import lax
from jax.experimental import pallas as pl
from jax.experimental.pallas import tpu as pltpu
from jax.experimental.pallas import tpu_sc as plsc

_NC = 2
_NS = 16
_NW = _NC * _NS
_LANES = 16
_CH = 8
_HALF = 32


def kernel(x, pos_embedding):
    b, s, d = x.shape
    rows_per_w = s // _NW
    n_chunks = rows_per_w // _CH
    n_iters = n_chunks // 2
    groups = d // _LANES
    mesh = plsc.VectorSubcoreMesh(core_axis_name="c", subcore_axis_name="s")

    @functools.partial(
        pl.kernel,
        mesh=mesh,
        out_type=jax.ShapeDtypeStruct((b, s, d), jnp.float32),
        scratch_types=[
            pltpu.VMEM((2, _CH, d), jnp.float32),
            pltpu.VMEM((2, b, _CH, d), jnp.float32),
            pltpu.SemaphoreType.DMA((2,)),
            pltpu.SemaphoreType.DMA((2,)),
            pltpu.SemaphoreType.DMA((2,)),
        ],
    )
    def k(x_hbm, pos_hbm, out_hbm, pos_v, x_v, sem_in, sem_out, sem_pos):
        wid = lax.axis_index("s") * _NC + lax.axis_index("c")
        row_base = wid * rows_per_w

        pltpu.async_copy(pos_hbm.at[pl.ds(row_base, _CH)], pos_v.at[0],
                         sem_pos.at[0])
        pltpu.async_copy(x_hbm.at[:, pl.ds(row_base, _CH)], x_v.at[0],
                         sem_in.at[0])

        def iter_body(cp, _):
            for k_ in range(2):
                xb = k_
                ob = (k_ + 1) % 2
                rows = row_base + (cp * 2 + k_) * _CH
                nxt = rows + _CH

                pltpu.async_copy(pos_hbm.at[pl.ds(nxt, _CH)],
                                 pos_v.at[ob], sem_pos.at[ob])

                if k_ == 0:
                    @pl.when(cp > 0)
                    def _():
                        pltpu.make_async_copy(
                            x_v.at[ob],
                            out_hbm.at[:, pl.ds(rows - _CH, _CH)],
                            sem_out.at[ob]).wait()
                    pltpu.async_copy(x_hbm.at[:, pl.ds(nxt, _CH)],
                                     x_v.at[ob], sem_in.at[ob])
                else:
                    pltpu.make_async_copy(
                        x_v.at[ob], out_hbm.at[:, pl.ds(rows - _CH, _CH)],
                        sem_out.at[ob]).wait()

                    @pl.when(cp + 1 < n_iters)
                    def _():
                        pltpu.async_copy(x_hbm.at[:, pl.ds(nxt, _CH)],
                                         x_v.at[ob], sem_in.at[ob])

                pltpu.make_async_copy(
                    x_hbm.at[:, pl.ds(rows, _CH)], x_v.at[xb],
                    sem_in.at[xb]).wait()
                pltpu.make_async_copy(
                    pos_hbm.at[pl.ds(rows, _CH)], pos_v.at[xb],
                    sem_pos.at[xb]).wait()

                def row_body(r, _):
                    for h in range(groups // _HALF):
                        base = h * _HALF * _LANES
                        pos_regs = [
                            pos_v[xb, r, pl.ds(base + g * _LANES, _LANES)]
                            for g in range(_HALF)
                        ]
                        for bi in range(b):
                            for g in range(_HALF):
                                o = base + g * _LANES
                                x_v[xb, bi, r, pl.ds(o, _LANES)] = (
                                    x_v[xb, bi, r, pl.ds(o, _LANES)]
                                    + pos_regs[g])
                    return 0

                hc = _CH // 2
                for hh in range(2):
                    lax.fori_loop(hh * hc, (hh + 1) * hc, row_body, 0)
                    pltpu.async_copy(
                        x_v.at[xb, :, pl.ds(hh * hc, hc)],
                        out_hbm.at[:, pl.ds(rows + hh * hc, hc)],
                        sem_out.at[xb])
            return 0

        lax.fori_loop(0, n_iters, iter_body, 0)

        last = row_base + (n_chunks - 1) * _CH
        pltpu.make_async_copy(
            x_v.at[1], out_hbm.at[:, pl.ds(last, _CH)], sem_out.at[1]).wait()
        pltpu.make_async_copy(
            pos_hbm.at[pl.ds(row_base, _CH)], pos_v.at[0],
            sem_pos.at[0]).wait()

    return k(x, pos_embedding)
